# Initial kernel scaffold; baseline (speedup 1.0000x reference)
import jax
import jax.numpy as jnp
from jax.experimental import pallas as pl


def kernel(z_e, embedding):
    raise NotImplementedError("write your pallas kernel here")


# speedup vs baseline: 1.0725x; 1.0725x over previous
import functools

import jax
import jax.numpy as jnp
from jax import lax
from jax.experimental import pallas as pl
from jax.experimental.pallas import tpu as pltpu
from jax.experimental.pallas import tpu_sc as plsc

KC = 8192
DC = 64
BETA_C = 0.25

TN = 512
TK = 2048
NB = 16384 // TN
KB = KC // TK


def _vq_argmin_body(flat_ref, embt_ref, fnorm_ref, enorm_ref,
                    idx_ref, bval_ref, best_val, best_idx):
    kb = pl.program_id(1)

    fb = flat_ref[...]
    et = embt_ref[...]
    mm = lax.dot_general(fb, et, (((1,), (0,)), ((), ())),
                         preferred_element_type=jnp.float32)
    scores = (fnorm_ref[...] + enorm_ref[...]) - 2.0 * mm

    tmin = jnp.min(scores, axis=1, keepdims=True)
    col = lax.broadcasted_iota(jnp.int32, (TN, TK), 1) + kb * TK
    tidx = jnp.min(jnp.where(scores == tmin, col, jnp.int32(2**30)),
                   axis=1, keepdims=True)

    @pl.when(kb == 0)
    def _():
        best_val[...] = jnp.full((TN, 1), jnp.inf, jnp.float32)
        best_idx[...] = jnp.zeros((TN, 1), jnp.int32)

    bv = best_val[...]
    better = tmin < bv
    best_idx[...] = jnp.where(better, tidx, best_idx[...])
    best_val[...] = jnp.where(better, tmin, bv)

    @pl.when(kb == KB - 1)
    def _():
        idx_ref[0, :, :] = best_idx[...]
        bval_ref[0, :, :] = best_val[...]


def _vq_argmin(flat, embt, fnorm, enorm):
    return pl.pallas_call(
        _vq_argmin_body,
        grid=(NB, KB),
        in_specs=[
            pl.BlockSpec((TN, DC), lambda i, k: (i, 0)),
            pl.BlockSpec((DC, TK), lambda i, k: (0, k)),
            pl.BlockSpec((TN, 1), lambda i, k: (i, 0)),
            pl.BlockSpec((1, TK), lambda i, k: (0, k)),
        ],
        out_specs=[
            pl.BlockSpec((1, TN, 1), lambda i, k: (i, 0, 0)),
            pl.BlockSpec((1, TN, 1), lambda i, k: (i, 0, 0)),
        ],
        out_shape=[
            jax.ShapeDtypeStruct((NB, TN, 1), jnp.int32),
            jax.ShapeDtypeStruct((NB, TN, 1), jnp.float32),
        ],
        scratch_shapes=[
            pltpu.VMEM((TN, 1), jnp.float32),
            pltpu.VMEM((TN, 1), jnp.int32),
        ],
        compiler_params=pltpu.CompilerParams(
            dimension_semantics=("arbitrary", "arbitrary")),
    )(flat, embt, fnorm, enorm)


_SC_GATHER_CACHE = []


def _build_sc_gather():
    info = plsc.get_sparse_core_info()
    nc = info.num_cores
    nw = nc * info.num_subcores
    bpw = 16384 // nw

    @functools.partial(
        pl.kernel,
        mesh=plsc.VectorSubcoreMesh(core_axis_name="c", subcore_axis_name="s"),
        out_type=jax.ShapeDtypeStruct((16384, DC), jnp.float32),
        scratch_types=[
            pltpu.VMEM((bpw,), jnp.int32),
            pltpu.VMEM((bpw, DC), jnp.float32),
            pltpu.SemaphoreType.DMA,
        ],
        compiler_params=pltpu.CompilerParams(use_tc_tiling_on_sc=False),
    )
    def gather_rows(table_hbm, idx_hbm, out_hbm, idx_v, rows_v, sem):
        wid = lax.axis_index("s") * nc + lax.axis_index("c")
        base = wid * bpw
        pltpu.sync_copy(idx_hbm.at[pl.ds(base, bpw)], idx_v)
        pltpu.async_copy(table_hbm.at[idx_v], rows_v, sem).wait()
        pltpu.sync_copy(rows_v, out_hbm.at[pl.ds(base, bpw)])

    return gather_rows


def _sc_gather_rows(table, indices):
    if not _SC_GATHER_CACHE:
        _SC_GATHER_CACHE.append(_build_sc_gather())
    return _SC_GATHER_CACHE[0](table, indices)


def kernel(z_e, embedding):
    B, Dc, H, W = z_e.shape
    N = B * H * W
    flat = jnp.transpose(z_e, (0, 2, 3, 1)).reshape(N, Dc)
    fnorm = jnp.sum(flat ** 2, axis=1, keepdims=True)
    enorm = jnp.sum(embedding ** 2, axis=1).reshape(1, KC)
    embt = embedding.T

    idx3, bv3 = _vq_argmin(flat, embt, fnorm, enorm)
    indices = idx3.reshape(N)

    z_q_flat = _sc_gather_rows(embedding, indices)
    z_q = jnp.transpose(z_q_flat.reshape(B, H, W, Dc), (0, 3, 1, 2))

    commitment_loss = BETA_C * (jnp.sum(bv3) / (N * Dc))
    z_q_st = z_e + (z_q - z_e)
    return (z_q_st, indices.reshape(B, H, W), commitment_loss)
